# baseline (speedup 1.0000x reference)
import dataclasses
import functools

import jax
import jax.numpy as jnp
from jax import lax
from jax.experimental import pallas as pl
from jax.experimental.pallas import tpu as pltpu
from jax.experimental.pallas import tpu_sc as plsc

_NC = 2
_NS = 16
_NW = _NC * _NS


def _sc_compiler_params():
    cp = pltpu.CompilerParams()
    if "needs_layout_passes" in pltpu.CompilerParams.__dataclass_fields__:
        cp = dataclasses.replace(cp, needs_layout_passes=False)
    return cp


def _sc_hist(src, dst, n_nodes):
    e = src.shape[0]
    epw = e // _NW
    ch = 400
    mesh = plsc.VectorSubcoreMesh(core_axis_name="c", subcore_axis_name="s")

    @functools.partial(
        pl.kernel,
        out_type=[jax.ShapeDtypeStruct((_NW, n_nodes), jnp.float32),
                  jax.ShapeDtypeStruct((_NW, n_nodes), jnp.float32)],
        mesh=mesh,
        scratch_types=[pltpu.VMEM((ch,), jnp.int32),
                       pltpu.VMEM((n_nodes,), jnp.float32),
                       pltpu.VMEM((n_nodes,), jnp.float32)],
        compiler_params=_sc_compiler_params(),
    )
    def hist_kernel(src_hbm, dst_hbm, hs_hbm, hd_hbm, idx_v, hist_s, hist_d):
        c = lax.axis_index("c")
        s = lax.axis_index("s")
        w = c * _NS + s
        zero = jnp.zeros((16,), jnp.float32)
        ones = jnp.ones((16,), jnp.float32)

        @pl.loop(0, n_nodes // 16)
        def _(i):
            hist_s[pl.ds(i * 16, 16)] = zero
            hist_d[pl.ds(i * 16, 16)] = zero

        base = w * epw

        @pl.loop(0, epw // ch)
        def _(k):
            pltpu.sync_copy(src_hbm.at[pl.ds(base + k * ch, ch)], idx_v)

            @pl.loop(0, ch // 16)
            def _(j):
                idx = idx_v[pl.ds(j * 16, 16)]
                plsc.addupdate_scatter(hist_s, [idx], ones)

            pltpu.sync_copy(dst_hbm.at[pl.ds(base + k * ch, ch)], idx_v)

            @pl.loop(0, ch // 16)
            def _(j):
                idx = idx_v[pl.ds(j * 16, 16)]
                plsc.addupdate_scatter(hist_d, [idx], ones)

        pltpu.sync_copy(hist_s, hs_hbm.at[w])
        pltpu.sync_copy(hist_d, hd_hbm.at[w])

    return hist_kernel(src, dst)


def _sc_aggregate(xw, src, dst, n_out):
    n, d = xw.shape
    e = src.shape[0]
    ce = 128
    nch = e // (_NW * ce)
    rpw = (n // (8 * _NS)) * 8
    rem = n - rpw * _NS
    remo = n_out - rpw * _NS
    zr = rpw // 3
    zb = 16
    assert zr * 3 == rpw and rpw % zb == 0 and rem % zb == 0
    assert 0 <= remo <= rem and remo % 8 == 0
    assert nch % 2 == 0 and nch * ce * _NW == e
    mesh = plsc.VectorSubcoreMesh(core_axis_name="c", subcore_axis_name="s")

    @functools.partial(
        pl.kernel,
        out_type=jax.ShapeDtypeStruct((_NC, n_out, d), jnp.float32),
        mesh=mesh,
        scratch_types=[pltpu.VMEM((ce,), jnp.int32),
                       pltpu.VMEM((ce,), jnp.int32),
                       pltpu.VMEM((ce,), jnp.int32),
                       pltpu.VMEM((ce,), jnp.int32),
                       pltpu.VMEM((ce, d), jnp.float32),
                       pltpu.VMEM((ce, d), jnp.float32),
                       pltpu.VMEM((zb, d), jnp.float32),
                       pltpu.VMEM_SHARED((n, d), jnp.float32),
                       pltpu.SemaphoreType.DMA,
                       pltpu.SemaphoreType.DMA,
                       pltpu.SemaphoreType.DMA,
                       pltpu.SemaphoreType.DMA],
    )
    def agg_kernel(xw_hbm, src_hbm, dst_hbm, out_hbm,
                   idx_s0, idx_d0, idx_s1, idx_d1,
                   rows0, rows1, zbuf, agg_sh, semg0, semg1, semi0, semi1):
        c = lax.axis_index("c")
        s = lax.axis_index("s")
        w = c * _NS + s
        zero = jnp.zeros((16,), jnp.float32)

        @pl.loop(0, zb)
        def _(r):
            for g in range(d // 16):
                zbuf[r, pl.ds(g * 16, 16)] = zero

        @pl.loop(0, rpw // zb)
        def _(k):
            pltpu.sync_copy(zbuf, agg_sh.at[pl.ds(s * rpw + k * zb, zb)])

        @pl.when(s == _NS - 1)
        def _():
            for j in range(rem // zb):
                pltpu.sync_copy(zbuf,
                                agg_sh.at[pl.ds(_NS * rpw + j * zb, zb)])

        plsc.subcore_barrier()

        base = w * nch * ce

        def afetch(k, i_s, i_d, sem):
            pltpu.async_copy(src_hbm.at[pl.ds(base + k * ce, ce)], i_s, sem)
            pltpu.async_copy(dst_hbm.at[pl.ds(base + k * ce, ce)], i_d, sem)

        def wait_idx(i_s, i_d, sem):
            pltpu.make_async_copy(src_hbm.at[pl.ds(0, ce)], i_s, sem).wait()
            pltpu.make_async_copy(src_hbm.at[pl.ds(0, ce)], i_d, sem).wait()

        def wait_rows(r, sem):
            pltpu.make_async_copy(xw_hbm.at[idx_s0], r, sem).wait()

        afetch(0, idx_s0, idx_d0, semi0)
        wait_idx(idx_s0, idx_d0, semi0)
        pltpu.async_copy(xw_hbm.at[idx_s0], rows0, semg0)
        afetch(1, idx_s1, idx_d1, semi1)

        @pl.loop(0, nch // 2 - 1)
        def _(i):
            k = 2 * i
            wait_idx(idx_s1, idx_d1, semi1)
            pltpu.async_copy(xw_hbm.at[idx_s1], rows1, semg1)
            wait_rows(rows0, semg0)
            pltpu.sync_copy(rows0, agg_sh.at[idx_d0], add=True)
            afetch(k + 2, idx_s0, idx_d0, semi0)
            wait_idx(idx_s0, idx_d0, semi0)
            pltpu.async_copy(xw_hbm.at[idx_s0], rows0, semg0)
            wait_rows(rows1, semg1)
            pltpu.sync_copy(rows1, agg_sh.at[idx_d1], add=True)

            @pl.when(k + 3 < nch)
            def _():
                afetch(k + 3, idx_s1, idx_d1, semi1)

        wait_idx(idx_s1, idx_d1, semi1)
        pltpu.async_copy(xw_hbm.at[idx_s1], rows1, semg1)
        wait_rows(rows0, semg0)
        pltpu.sync_copy(rows0, agg_sh.at[idx_d0], add=True)
        wait_rows(rows1, semg1)
        pltpu.sync_copy(rows1, agg_sh.at[idx_d1], add=True)

        plsc.subcore_barrier()
        for k in range(3):
            off = s * rpw + k * zr
            pltpu.sync_copy(agg_sh.at[pl.ds(off, zr)],
                            out_hbm.at[c, pl.ds(off, zr)])

        if remo:
            @pl.when(s == _NS - 1)
            def _():
                pltpu.sync_copy(agg_sh.at[pl.ds(_NS * rpw, remo)],
                                out_hbm.at[c, pl.ds(_NS * rpw, remo)])

    return agg_kernel(xw, src, dst)


def _norms_body(hs_ref, hd_ref, ns_ref, nd_ref):
    ones = jnp.ones((_NW, 1), jnp.float32)
    dims = (((0,), (0,)), ((), ()))
    deg_s = lax.dot_general(hs_ref[...], ones, dims,
                            preferred_element_type=jnp.float32)
    deg_d = lax.dot_general(hd_ref[...], ones, dims,
                            preferred_element_type=jnp.float32)
    ns_ref[...] = lax.rsqrt(jnp.maximum(deg_s, 1.0))
    nd_ref[...] = lax.rsqrt(jnp.maximum(deg_d, 1.0))


def _tc_norms(hs, hd):
    n = hs.shape[1]
    out = jax.ShapeDtypeStruct((n, 1), jnp.float32)
    return pl.pallas_call(
        _norms_body,
        out_shape=[out, out],
    )(hs, hd)


def _mm_body(h_ref, ns_ref, w_ref, out_ref):
    x = h_ref[...] * ns_ref[...]
    out_ref[...] = lax.dot_general(
        x, w_ref[...], (((1,), (0,)), ((), ())),
        preferred_element_type=jnp.float32,
        precision=lax.Precision.HIGHEST)


def _tc_matmul(h, ns, w):
    n, d_in = h.shape
    dp = w.shape[1]
    blk = 1000
    return pl.pallas_call(
        _mm_body,
        grid=(n // blk,),
        in_specs=[pl.BlockSpec((blk, d_in), lambda i: (i, 0)),
                  pl.BlockSpec((blk, 1), lambda i: (i, 0)),
                  pl.BlockSpec((d_in, dp), lambda i: (0, 0))],
        out_specs=pl.BlockSpec((blk, dp), lambda i: (i, 0)),
        out_shape=jax.ShapeDtypeStruct((n, dp), jnp.float32),
    )(h, ns, w)


def _post_compute(agg_ref, nd_ref, b_ref, g_ref, be_ref, dv):
    a = agg_ref[0] + agg_ref[1]
    x = a * nd_ref[...] + b_ref[...]
    dp = x.shape[1]
    if dv == dp:
        mu = jnp.mean(x, axis=1, keepdims=True)
        xc = x - mu
        var = jnp.mean(xc * xc, axis=1, keepdims=True)
    else:
        mask = (lax.broadcasted_iota(jnp.int32, x.shape, 1) < dv)
        xm = jnp.where(mask, x, 0.0)
        mu = jnp.sum(xm, axis=1, keepdims=True) / dv
        xc = x - mu
        xcm = jnp.where(mask, xc, 0.0)
        var = jnp.sum(xcm * xcm, axis=1, keepdims=True) / dv
    y = xc * lax.rsqrt(var + 1e-5) * g_ref[...] + be_ref[...]
    return jnp.maximum(y, 0.0)


def _post_body(agg_ref, nd_ref, b_ref, g_ref, be_ref, out_ref, *, dv):
    out_ref[...] = _post_compute(agg_ref, nd_ref, b_ref, g_ref, be_ref, dv)


def _postmm_body(agg_ref, nd_ref, b_ref, g_ref, be_ref, ns_ref, w_ref,
                 out_ref, *, dv):
    h = _post_compute(agg_ref, nd_ref, b_ref, g_ref, be_ref, dv)
    x = h * ns_ref[...]
    out_ref[...] = lax.dot_general(
        x, w_ref[...], (((1,), (0,)), ((), ())),
        preferred_element_type=jnp.float32,
        precision=lax.Precision.HIGHEST)


def _tc_postmm(agg, nd, b, g, be, ns, w, dv):
    _, n, dp = agg.shape
    dp2 = w.shape[1]
    blk = 1000
    vec = pl.BlockSpec((1, dp), lambda i: (0, 0))
    return pl.pallas_call(
        functools.partial(_postmm_body, dv=dv),
        grid=(n // blk,),
        in_specs=[pl.BlockSpec((_NC, blk, dp), lambda i: (0, i, 0)),
                  pl.BlockSpec((blk, 1), lambda i: (i, 0)),
                  vec, vec, vec,
                  pl.BlockSpec((blk, 1), lambda i: (i, 0)),
                  pl.BlockSpec((dp, dp2), lambda i: (0, 0))],
        out_specs=pl.BlockSpec((blk, dp2), lambda i: (i, 0)),
        out_shape=jax.ShapeDtypeStruct((n, dp2), jnp.float32),
    )(agg, nd, b.reshape(1, dp), g.reshape(1, dp), be.reshape(1, dp), ns, w)


def _tc_post(agg, nd, b, g, be, dv):
    _, n, dp = agg.shape
    blk = 1000
    vec = pl.BlockSpec((1, dp), lambda i: (0, 0))
    return pl.pallas_call(
        functools.partial(_post_body, dv=dv),
        grid=(n // blk,),
        in_specs=[pl.BlockSpec((_NC, blk, dp), lambda i: (0, i, 0)),
                  pl.BlockSpec((blk, 1), lambda i: (i, 0)),
                  vec, vec, vec],
        out_specs=pl.BlockSpec((blk, dp), lambda i: (i, 0)),
        out_shape=jax.ShapeDtypeStruct((n, dp), jnp.float32),
    )(agg, nd, b.reshape(1, dp), g.reshape(1, dp), be.reshape(1, dp))


def kernel(features, W1, b1, g1, be1, W2, b2, g2, be2, W3, b3, g3, be3,
           edge_index):
    n = features.shape[0]
    src = edge_index[0]
    dst = edge_index[1]
    n_cls = W3.shape[1]
    dp3 = 128
    w3p = jnp.pad(W3, ((0, 0), (0, dp3 - n_cls)))
    b3p = jnp.pad(b3, (0, dp3 - n_cls))
    g3p = jnp.pad(g3, (0, dp3 - n_cls))
    be3p = jnp.pad(be3, (0, dp3 - n_cls))

    hs, hd = _sc_hist(src, dst, n)
    ns, nd = _tc_norms(hs, hd)

    e = src.shape[0]
    epc = _NW * 128
    e_pad = -(-e // (2 * epc)) * 2 * epc
    n_pad = n + 16
    pad_i = jnp.full((e_pad - e,), n, dtype=src.dtype)
    src_p = jnp.concatenate([src, pad_i])
    dst_p = jnp.concatenate([dst, pad_i])
    zrows = jnp.zeros((n_pad - n, 128), jnp.float32)

    def agg_of(xw):
        return _sc_aggregate(jnp.concatenate([xw, zrows]), src_p, dst_p, n)

    xw1 = _tc_matmul(features, ns, W1)
    agg1 = agg_of(xw1)
    xw2 = _tc_postmm(agg1, nd, b1, g1, be1, ns, W2, 128)
    agg2 = agg_of(xw2)
    xw3 = _tc_postmm(agg2, nd, b2, g2, be2, ns, w3p, 128)
    agg3 = agg_of(xw3)
    h = _tc_post(agg3, nd, b3p, g3p, be3p, n_cls)
    return h[:, :n_cls]


# speedup vs baseline: 3.1614x; 3.1614x over previous
import dataclasses
import functools

import jax
import jax.numpy as jnp
from jax import lax
from jax.experimental import pallas as pl
from jax.experimental.pallas import tpu as pltpu
from jax.experimental.pallas import tpu_sc as plsc

_NC = 2
_NS = 16
_NW = _NC * _NS


def _sc_compiler_params():
    cp = pltpu.CompilerParams()
    if "needs_layout_passes" in pltpu.CompilerParams.__dataclass_fields__:
        cp = dataclasses.replace(cp, needs_layout_passes=False)
    return cp


def _sc_hist(src, dst, n_nodes):
    e = src.shape[0]
    epw = e // _NW
    ch = 400
    mesh = plsc.VectorSubcoreMesh(core_axis_name="c", subcore_axis_name="s")

    @functools.partial(
        pl.kernel,
        out_type=[jax.ShapeDtypeStruct((_NW, n_nodes), jnp.float32),
                  jax.ShapeDtypeStruct((_NW, n_nodes), jnp.float32)],
        mesh=mesh,
        scratch_types=[pltpu.VMEM((ch,), jnp.int32),
                       pltpu.VMEM((n_nodes,), jnp.float32),
                       pltpu.VMEM((n_nodes,), jnp.float32)],
        compiler_params=_sc_compiler_params(),
    )
    def hist_kernel(src_hbm, dst_hbm, hs_hbm, hd_hbm, idx_v, hist_s, hist_d):
        c = lax.axis_index("c")
        s = lax.axis_index("s")
        w = c * _NS + s
        zero = jnp.zeros((16,), jnp.float32)
        ones = jnp.ones((16,), jnp.float32)

        @pl.loop(0, n_nodes // 16)
        def _(i):
            hist_s[pl.ds(i * 16, 16)] = zero
            hist_d[pl.ds(i * 16, 16)] = zero

        base = w * epw

        @pl.loop(0, epw // ch)
        def _(k):
            pltpu.sync_copy(src_hbm.at[pl.ds(base + k * ch, ch)], idx_v)

            @pl.loop(0, ch // 16)
            def _(j):
                idx = idx_v[pl.ds(j * 16, 16)]
                plsc.addupdate_scatter(hist_s, [idx], ones)

            pltpu.sync_copy(dst_hbm.at[pl.ds(base + k * ch, ch)], idx_v)

            @pl.loop(0, ch // 16)
            def _(j):
                idx = idx_v[pl.ds(j * 16, 16)]
                plsc.addupdate_scatter(hist_d, [idx], ones)

        pltpu.sync_copy(hist_s, hs_hbm.at[w])
        pltpu.sync_copy(hist_d, hd_hbm.at[w])

    return hist_kernel(src, dst)


def _sc_aggregate(xw, src, dst):
    n, d = xw.shape
    e = src.shape[0]
    ce = 128
    nch = (e // (_NW * ce)) & ~1
    extra = (e - _NW * nch * ce) // ce
    rpw = (n // (8 * _NS)) * 8
    rem = n - rpw * _NS
    zr = rpw // 3
    zb = 16
    assert zr * 3 == rpw and rpw % zb == 0 and rem % zb == 0
    assert _NW * nch * ce + extra * ce == e and 0 <= extra < _NW
    mesh = plsc.VectorSubcoreMesh(core_axis_name="c", subcore_axis_name="s")

    @functools.partial(
        pl.kernel,
        out_type=jax.ShapeDtypeStruct((_NC, n, d), jnp.float32),
        mesh=mesh,
        scratch_types=[pltpu.VMEM((ce,), jnp.int32),
                       pltpu.VMEM((ce,), jnp.int32),
                       pltpu.VMEM((ce,), jnp.int32),
                       pltpu.VMEM((ce,), jnp.int32),
                       pltpu.VMEM((ce, d), jnp.float32),
                       pltpu.VMEM((ce, d), jnp.float32),
                       pltpu.VMEM((zb, d), jnp.float32),
                       pltpu.VMEM_SHARED((n, d), jnp.float32),
                       pltpu.SemaphoreType.DMA,
                       pltpu.SemaphoreType.DMA,
                       pltpu.SemaphoreType.DMA,
                       pltpu.SemaphoreType.DMA],
    )
    def agg_kernel(xw_hbm, src_hbm, dst_hbm, out_hbm,
                   idx_s0, idx_d0, idx_s1, idx_d1,
                   rows0, rows1, zbuf, agg_sh, semg0, semg1, semi0, semi1):
        c = lax.axis_index("c")
        s = lax.axis_index("s")
        w = c * _NS + s
        zero = jnp.zeros((16,), jnp.float32)

        @pl.loop(0, zb)
        def _(r):
            for g in range(d // 16):
                zbuf[r, pl.ds(g * 16, 16)] = zero

        @pl.loop(0, rpw // zb)
        def _(k):
            pltpu.sync_copy(zbuf, agg_sh.at[pl.ds(s * rpw + k * zb, zb)])

        @pl.when(s == _NS - 1)
        def _():
            for j in range(rem // zb):
                pltpu.sync_copy(zbuf,
                                agg_sh.at[pl.ds(_NS * rpw + j * zb, zb)])

        plsc.subcore_barrier()

        base = w * nch * ce

        def afetch(k, i_s, i_d, sem):
            pltpu.async_copy(src_hbm.at[pl.ds(base + k * ce, ce)], i_s, sem)
            pltpu.async_copy(dst_hbm.at[pl.ds(base + k * ce, ce)], i_d, sem)

        def wait_idx(i_s, i_d, sem):
            pltpu.make_async_copy(src_hbm.at[pl.ds(0, ce)], i_s, sem).wait()
            pltpu.make_async_copy(src_hbm.at[pl.ds(0, ce)], i_d, sem).wait()

        def wait_rows(r, sem):
            pltpu.make_async_copy(xw_hbm.at[idx_s0], r, sem).wait()

        afetch(0, idx_s0, idx_d0, semi0)
        wait_idx(idx_s0, idx_d0, semi0)
        pltpu.async_copy(xw_hbm.at[idx_s0], rows0, semg0)
        afetch(1, idx_s1, idx_d1, semi1)

        @pl.loop(0, nch // 2 - 1)
        def _(i):
            k = 2 * i
            wait_idx(idx_s1, idx_d1, semi1)
            pltpu.async_copy(xw_hbm.at[idx_s1], rows1, semg1)
            wait_rows(rows0, semg0)
            pltpu.sync_copy(rows0, agg_sh.at[idx_d0], add=True)
            afetch(k + 2, idx_s0, idx_d0, semi0)
            wait_idx(idx_s0, idx_d0, semi0)
            pltpu.async_copy(xw_hbm.at[idx_s0], rows0, semg0)
            wait_rows(rows1, semg1)
            pltpu.sync_copy(rows1, agg_sh.at[idx_d1], add=True)

            @pl.when(k + 3 < nch)
            def _():
                afetch(k + 3, idx_s1, idx_d1, semi1)

        wait_idx(idx_s1, idx_d1, semi1)
        pltpu.async_copy(xw_hbm.at[idx_s1], rows1, semg1)
        wait_rows(rows0, semg0)
        pltpu.sync_copy(rows0, agg_sh.at[idx_d0], add=True)
        wait_rows(rows1, semg1)
        pltpu.sync_copy(rows1, agg_sh.at[idx_d1], add=True)

        if extra:
            @pl.when(w < extra)
            def _():
                xoff = _NW * nch * ce + w * ce
                pltpu.async_copy(src_hbm.at[pl.ds(xoff, ce)], idx_s0, semi0)
                pltpu.async_copy(dst_hbm.at[pl.ds(xoff, ce)], idx_d0, semi0)
                wait_idx(idx_s0, idx_d0, semi0)
                pltpu.async_copy(xw_hbm.at[idx_s0], rows0, semg0)
                wait_rows(rows0, semg0)
                pltpu.sync_copy(rows0, agg_sh.at[idx_d0], add=True)

        plsc.subcore_barrier()
        for k in range(3):
            off = s * rpw + k * zr
            pltpu.sync_copy(agg_sh.at[pl.ds(off, zr)],
                            out_hbm.at[c, pl.ds(off, zr)])

        if rem:
            @pl.when(s == _NS - 1)
            def _():
                pltpu.sync_copy(agg_sh.at[pl.ds(_NS * rpw, rem)],
                                out_hbm.at[c, pl.ds(_NS * rpw, rem)])

    return agg_kernel(xw, src, dst)


def _norms_body(hs_ref, hd_ref, ns_ref, nd_ref):
    ones = jnp.ones((_NW, 1), jnp.float32)
    dims = (((0,), (0,)), ((), ()))
    deg_s = lax.dot_general(hs_ref[...], ones, dims,
                            preferred_element_type=jnp.float32)
    deg_d = lax.dot_general(hd_ref[...], ones, dims,
                            preferred_element_type=jnp.float32)
    ns_ref[...] = lax.rsqrt(jnp.maximum(deg_s, 1.0))
    nd_ref[...] = lax.rsqrt(jnp.maximum(deg_d, 1.0))


def _tc_norms(hs, hd):
    n = hs.shape[1]
    out = jax.ShapeDtypeStruct((n, 1), jnp.float32)
    return pl.pallas_call(
        _norms_body,
        out_shape=[out, out],
    )(hs, hd)


def _mm_body(h_ref, ns_ref, w_ref, out_ref):
    x = h_ref[...] * ns_ref[...]
    out_ref[...] = lax.dot_general(
        x, w_ref[...], (((1,), (0,)), ((), ())),
        preferred_element_type=jnp.float32,
        precision=lax.Precision.HIGHEST)


def _tc_matmul(h, ns, w):
    n, d_in = h.shape
    dp = w.shape[1]
    blk = 1000
    return pl.pallas_call(
        _mm_body,
        grid=(n // blk,),
        in_specs=[pl.BlockSpec((blk, d_in), lambda i: (i, 0)),
                  pl.BlockSpec((blk, 1), lambda i: (i, 0)),
                  pl.BlockSpec((d_in, dp), lambda i: (0, 0))],
        out_specs=pl.BlockSpec((blk, dp), lambda i: (i, 0)),
        out_shape=jax.ShapeDtypeStruct((n, dp), jnp.float32),
    )(h, ns, w)


def _post_compute(agg_ref, nd_ref, b_ref, g_ref, be_ref, dv):
    a = agg_ref[0] + agg_ref[1]
    x = a * nd_ref[...] + b_ref[...]
    dp = x.shape[1]
    if dv == dp:
        mu = jnp.mean(x, axis=1, keepdims=True)
        xc = x - mu
        var = jnp.mean(xc * xc, axis=1, keepdims=True)
    else:
        mask = (lax.broadcasted_iota(jnp.int32, x.shape, 1) < dv)
        xm = jnp.where(mask, x, 0.0)
        mu = jnp.sum(xm, axis=1, keepdims=True) / dv
        xc = x - mu
        xcm = jnp.where(mask, xc, 0.0)
        var = jnp.sum(xcm * xcm, axis=1, keepdims=True) / dv
    y = xc * lax.rsqrt(var + 1e-5) * g_ref[...] + be_ref[...]
    return jnp.maximum(y, 0.0)


def _post_body(agg_ref, nd_ref, b_ref, g_ref, be_ref, out_ref, *, dv):
    out_ref[...] = _post_compute(agg_ref, nd_ref, b_ref, g_ref, be_ref, dv)


def _postmm_body(agg_ref, nd_ref, b_ref, g_ref, be_ref, ns_ref, w_ref,
                 out_ref, *, dv):
    h = _post_compute(agg_ref, nd_ref, b_ref, g_ref, be_ref, dv)
    x = h * ns_ref[...]
    out_ref[...] = lax.dot_general(
        x, w_ref[...], (((1,), (0,)), ((), ())),
        preferred_element_type=jnp.float32,
        precision=lax.Precision.HIGHEST)


def _tc_postmm(agg, nd, b, g, be, ns, w, dv):
    _, n, dp = agg.shape
    dp2 = w.shape[1]
    blk = 1000
    vec = pl.BlockSpec((1, dp), lambda i: (0, 0))
    return pl.pallas_call(
        functools.partial(_postmm_body, dv=dv),
        grid=(n // blk,),
        in_specs=[pl.BlockSpec((_NC, blk, dp), lambda i: (0, i, 0)),
                  pl.BlockSpec((blk, 1), lambda i: (i, 0)),
                  vec, vec, vec,
                  pl.BlockSpec((blk, 1), lambda i: (i, 0)),
                  pl.BlockSpec((dp, dp2), lambda i: (0, 0))],
        out_specs=pl.BlockSpec((blk, dp2), lambda i: (i, 0)),
        out_shape=jax.ShapeDtypeStruct((n, dp2), jnp.float32),
    )(agg, nd, b.reshape(1, dp), g.reshape(1, dp), be.reshape(1, dp), ns, w)


def _tc_post(agg, nd, b, g, be, dv):
    _, n, dp = agg.shape
    blk = 1000
    vec = pl.BlockSpec((1, dp), lambda i: (0, 0))
    return pl.pallas_call(
        functools.partial(_post_body, dv=dv),
        grid=(n // blk,),
        in_specs=[pl.BlockSpec((_NC, blk, dp), lambda i: (0, i, 0)),
                  pl.BlockSpec((blk, 1), lambda i: (i, 0)),
                  vec, vec, vec],
        out_specs=pl.BlockSpec((blk, dp), lambda i: (i, 0)),
        out_shape=jax.ShapeDtypeStruct((n, dp), jnp.float32),
    )(agg, nd, b.reshape(1, dp), g.reshape(1, dp), be.reshape(1, dp))


def kernel(features, W1, b1, g1, be1, W2, b2, g2, be2, W3, b3, g3, be3,
           edge_index):
    n = features.shape[0]
    src = edge_index[0]
    dst = edge_index[1]
    n_cls = W3.shape[1]
    dp3 = 128
    w3p = jnp.pad(W3, ((0, 0), (0, dp3 - n_cls)))
    b3p = jnp.pad(b3, (0, dp3 - n_cls))
    g3p = jnp.pad(g3, (0, dp3 - n_cls))
    be3p = jnp.pad(be3, (0, dp3 - n_cls))

    hs, hd = _sc_hist(src, dst, n)
    ns, nd = _tc_norms(hs, hd)

    def agg_of(xw):
        return _sc_aggregate(xw, src, dst)

    xw1 = _tc_matmul(features, ns, W1)
    agg1 = agg_of(xw1)
    xw2 = _tc_postmm(agg1, nd, b1, g1, be1, ns, W2, 128)
    agg2 = agg_of(xw2)
    xw3 = _tc_postmm(agg2, nd, b2, g2, be2, ns, w3p, 128)
    agg3 = agg_of(xw3)
    h = _tc_post(agg3, nd, b3p, g3p, be3p, n_cls)
    return h[:, :n_cls]


# speedup vs baseline: 3.4030x; 1.0764x over previous
import dataclasses
import functools

import jax
import jax.numpy as jnp
from jax import lax
from jax.experimental import pallas as pl
from jax.experimental.pallas import tpu as pltpu
from jax.experimental.pallas import tpu_sc as plsc

_NC = 2
_NS = 16
_NW = _NC * _NS


def _sc_compiler_params():
    cp = pltpu.CompilerParams()
    if "needs_layout_passes" in pltpu.CompilerParams.__dataclass_fields__:
        cp = dataclasses.replace(cp, needs_layout_passes=False)
    return cp


def _sc_hist(src, dst, n_nodes):
    e = src.shape[0]
    epw = e // _NW
    ch = 2000
    mesh = plsc.VectorSubcoreMesh(core_axis_name="c", subcore_axis_name="s")

    @functools.partial(
        pl.kernel,
        out_type=[jax.ShapeDtypeStruct((_NW, n_nodes), jnp.float32),
                  jax.ShapeDtypeStruct((_NW, n_nodes), jnp.float32)],
        mesh=mesh,
        scratch_types=[pltpu.VMEM((ch,), jnp.int32),
                       pltpu.VMEM((n_nodes,), jnp.float32),
                       pltpu.VMEM((n_nodes,), jnp.float32)],
        compiler_params=_sc_compiler_params(),
    )
    def hist_kernel(src_hbm, dst_hbm, hs_hbm, hd_hbm, idx_v, hist_s, hist_d):
        c = lax.axis_index("c")
        s = lax.axis_index("s")
        w = c * _NS + s
        zero = jnp.zeros((16,), jnp.float32)
        ones = jnp.ones((16,), jnp.float32)

        @pl.loop(0, n_nodes // 16)
        def _(i):
            hist_s[pl.ds(i * 16, 16)] = zero
            hist_d[pl.ds(i * 16, 16)] = zero

        base = w * epw

        @pl.loop(0, epw // ch)
        def _(k):
            pltpu.sync_copy(src_hbm.at[pl.ds(base + k * ch, ch)], idx_v)

            @pl.loop(0, ch // 16)
            def _(j):
                idx = idx_v[pl.ds(j * 16, 16)]
                plsc.addupdate_scatter(hist_s, [idx], ones)

            pltpu.sync_copy(dst_hbm.at[pl.ds(base + k * ch, ch)], idx_v)

            @pl.loop(0, ch // 16)
            def _(j):
                idx = idx_v[pl.ds(j * 16, 16)]
                plsc.addupdate_scatter(hist_d, [idx], ones)

        pltpu.sync_copy(hist_s, hs_hbm.at[w])
        pltpu.sync_copy(hist_d, hd_hbm.at[w])

    return hist_kernel(src, dst)


def _sc_aggregate(xw, src, dst):
    n, d = xw.shape
    e = src.shape[0]
    ce = 128
    nch = (e // (_NW * ce)) & ~1
    extra = (e - _NW * nch * ce) // ce
    rpw = (n // (8 * _NS)) * 8
    rem = n - rpw * _NS
    zr = rpw // 3
    zb = 16
    assert zr * 3 == rpw and rpw % zb == 0 and rem % zb == 0
    assert _NW * nch * ce + extra * ce == e and 0 <= extra < _NW
    mesh = plsc.VectorSubcoreMesh(core_axis_name="c", subcore_axis_name="s")

    @functools.partial(
        pl.kernel,
        out_type=jax.ShapeDtypeStruct((_NC, n, d), jnp.float32),
        mesh=mesh,
        scratch_types=[pltpu.VMEM((ce,), jnp.int32),
                       pltpu.VMEM((ce,), jnp.int32),
                       pltpu.VMEM((ce,), jnp.int32),
                       pltpu.VMEM((ce,), jnp.int32),
                       pltpu.VMEM((ce, d), jnp.float32),
                       pltpu.VMEM((ce, d), jnp.float32),
                       pltpu.VMEM((zb, d), jnp.float32),
                       pltpu.VMEM_SHARED((n, d), jnp.float32),
                       pltpu.SemaphoreType.DMA,
                       pltpu.SemaphoreType.DMA,
                       pltpu.SemaphoreType.DMA,
                       pltpu.SemaphoreType.DMA],
    )
    def agg_kernel(xw_hbm, src_hbm, dst_hbm, out_hbm,
                   idx_s0, idx_d0, idx_s1, idx_d1,
                   rows0, rows1, zbuf, agg_sh, semg0, semg1, semi0, semi1):
        c = lax.axis_index("c")
        s = lax.axis_index("s")
        w = c * _NS + s
        zero = jnp.zeros((16,), jnp.float32)

        @pl.loop(0, zb)
        def _(r):
            for g in range(d // 16):
                zbuf[r, pl.ds(g * 16, 16)] = zero

        @pl.loop(0, rpw // zb)
        def _(k):
            pltpu.sync_copy(zbuf, agg_sh.at[pl.ds(s * rpw + k * zb, zb)])

        @pl.when(s == _NS - 1)
        def _():
            for j in range(rem // zb):
                pltpu.sync_copy(zbuf,
                                agg_sh.at[pl.ds(_NS * rpw + j * zb, zb)])

        plsc.subcore_barrier()

        base = w * nch * ce

        def afetch(k, i_s, i_d, sem):
            pltpu.async_copy(src_hbm.at[pl.ds(base + k * ce, ce)], i_s, sem)
            pltpu.async_copy(dst_hbm.at[pl.ds(base + k * ce, ce)], i_d, sem)

        def wait_idx(i_s, i_d, sem):
            pltpu.make_async_copy(src_hbm.at[pl.ds(0, ce)], i_s, sem).wait()
            pltpu.make_async_copy(src_hbm.at[pl.ds(0, ce)], i_d, sem).wait()

        def wait_rows(r, sem):
            pltpu.make_async_copy(xw_hbm.at[idx_s0], r, sem).wait()

        afetch(0, idx_s0, idx_d0, semi0)
        wait_idx(idx_s0, idx_d0, semi0)
        pltpu.async_copy(xw_hbm.at[idx_s0], rows0, semg0)
        afetch(1, idx_s1, idx_d1, semi1)

        @pl.loop(0, nch // 2 - 1)
        def _(i):
            k = 2 * i
            wait_idx(idx_s1, idx_d1, semi1)
            pltpu.async_copy(xw_hbm.at[idx_s1], rows1, semg1)
            wait_rows(rows0, semg0)
            pltpu.sync_copy(rows0, agg_sh.at[idx_d0], add=True)
            afetch(k + 2, idx_s0, idx_d0, semi0)
            wait_idx(idx_s0, idx_d0, semi0)
            pltpu.async_copy(xw_hbm.at[idx_s0], rows0, semg0)
            wait_rows(rows1, semg1)
            pltpu.sync_copy(rows1, agg_sh.at[idx_d1], add=True)

            @pl.when(k + 3 < nch)
            def _():
                afetch(k + 3, idx_s1, idx_d1, semi1)

        wait_idx(idx_s1, idx_d1, semi1)
        pltpu.async_copy(xw_hbm.at[idx_s1], rows1, semg1)
        wait_rows(rows0, semg0)
        pltpu.sync_copy(rows0, agg_sh.at[idx_d0], add=True)
        wait_rows(rows1, semg1)
        pltpu.sync_copy(rows1, agg_sh.at[idx_d1], add=True)

        if extra:
            @pl.when(w < extra)
            def _():
                xoff = _NW * nch * ce + w * ce
                pltpu.async_copy(src_hbm.at[pl.ds(xoff, ce)], idx_s0, semi0)
                pltpu.async_copy(dst_hbm.at[pl.ds(xoff, ce)], idx_d0, semi0)
                wait_idx(idx_s0, idx_d0, semi0)
                pltpu.async_copy(xw_hbm.at[idx_s0], rows0, semg0)
                wait_rows(rows0, semg0)
                pltpu.sync_copy(rows0, agg_sh.at[idx_d0], add=True)

        plsc.subcore_barrier()
        for k in range(3):
            off = s * rpw + k * zr
            pltpu.sync_copy(agg_sh.at[pl.ds(off, zr)],
                            out_hbm.at[c, pl.ds(off, zr)])

        if rem:
            @pl.when(s == _NS - 1)
            def _():
                pltpu.sync_copy(agg_sh.at[pl.ds(_NS * rpw, rem)],
                                out_hbm.at[c, pl.ds(_NS * rpw, rem)])

    return agg_kernel(xw, src, dst)


def _norms_body(hs_ref, hd_ref, ns_ref, nd_ref):
    ones = jnp.ones((_NW, 1), jnp.float32)
    dims = (((0,), (0,)), ((), ()))
    deg_s = lax.dot_general(hs_ref[...], ones, dims,
                            preferred_element_type=jnp.float32)
    deg_d = lax.dot_general(hd_ref[...], ones, dims,
                            preferred_element_type=jnp.float32)
    ns_ref[...] = lax.rsqrt(jnp.maximum(deg_s, 1.0))
    nd_ref[...] = lax.rsqrt(jnp.maximum(deg_d, 1.0))


def _tc_norms(hs, hd):
    n = hs.shape[1]
    out = jax.ShapeDtypeStruct((n, 1), jnp.float32)
    return pl.pallas_call(
        _norms_body,
        out_shape=[out, out],
    )(hs, hd)


def _mm_body(h_ref, ns_ref, w_ref, out_ref):
    x = h_ref[...] * ns_ref[...]
    out_ref[...] = lax.dot_general(
        x, w_ref[...], (((1,), (0,)), ((), ())),
        preferred_element_type=jnp.float32,
        precision=lax.Precision.DEFAULT)


def _tc_matmul(h, ns, w):
    n, d_in = h.shape
    dp = w.shape[1]
    blk = 1000
    return pl.pallas_call(
        _mm_body,
        grid=(n // blk,),
        in_specs=[pl.BlockSpec((blk, d_in), lambda i: (i, 0)),
                  pl.BlockSpec((blk, 1), lambda i: (i, 0)),
                  pl.BlockSpec((d_in, dp), lambda i: (0, 0))],
        out_specs=pl.BlockSpec((blk, dp), lambda i: (i, 0)),
        out_shape=jax.ShapeDtypeStruct((n, dp), jnp.float32),
    )(h, ns, w)


def _post_compute(agg_ref, nd_ref, b_ref, g_ref, be_ref, dv):
    a = agg_ref[0] + agg_ref[1]
    x = a * nd_ref[...] + b_ref[...]
    dp = x.shape[1]
    if dv == dp:
        mu = jnp.mean(x, axis=1, keepdims=True)
        xc = x - mu
        var = jnp.mean(xc * xc, axis=1, keepdims=True)
    else:
        mask = (lax.broadcasted_iota(jnp.int32, x.shape, 1) < dv)
        xm = jnp.where(mask, x, 0.0)
        mu = jnp.sum(xm, axis=1, keepdims=True) / dv
        xc = x - mu
        xcm = jnp.where(mask, xc, 0.0)
        var = jnp.sum(xcm * xcm, axis=1, keepdims=True) / dv
    y = xc * lax.rsqrt(var + 1e-5) * g_ref[...] + be_ref[...]
    return jnp.maximum(y, 0.0)


def _post_body(agg_ref, nd_ref, b_ref, g_ref, be_ref, out_ref, *, dv):
    out_ref[...] = _post_compute(agg_ref, nd_ref, b_ref, g_ref, be_ref, dv)


def _postmm_body(agg_ref, nd_ref, b_ref, g_ref, be_ref, ns_ref, w_ref,
                 out_ref, *, dv):
    h = _post_compute(agg_ref, nd_ref, b_ref, g_ref, be_ref, dv)
    x = h * ns_ref[...]
    out_ref[...] = lax.dot_general(
        x, w_ref[...], (((1,), (0,)), ((), ())),
        preferred_element_type=jnp.float32,
        precision=lax.Precision.DEFAULT)


def _tc_postmm(agg, nd, b, g, be, ns, w, dv):
    _, n, dp = agg.shape
    dp2 = w.shape[1]
    blk = 1000
    vec = pl.BlockSpec((1, dp), lambda i: (0, 0))
    return pl.pallas_call(
        functools.partial(_postmm_body, dv=dv),
        grid=(n // blk,),
        in_specs=[pl.BlockSpec((_NC, blk, dp), lambda i: (0, i, 0)),
                  pl.BlockSpec((blk, 1), lambda i: (i, 0)),
                  vec, vec, vec,
                  pl.BlockSpec((blk, 1), lambda i: (i, 0)),
                  pl.BlockSpec((dp, dp2), lambda i: (0, 0))],
        out_specs=pl.BlockSpec((blk, dp2), lambda i: (i, 0)),
        out_shape=jax.ShapeDtypeStruct((n, dp2), jnp.float32),
    )(agg, nd, b.reshape(1, dp), g.reshape(1, dp), be.reshape(1, dp), ns, w)


def _tc_post(agg, nd, b, g, be, dv):
    _, n, dp = agg.shape
    blk = 1000
    vec = pl.BlockSpec((1, dp), lambda i: (0, 0))
    return pl.pallas_call(
        functools.partial(_post_body, dv=dv),
        grid=(n // blk,),
        in_specs=[pl.BlockSpec((_NC, blk, dp), lambda i: (0, i, 0)),
                  pl.BlockSpec((blk, 1), lambda i: (i, 0)),
                  vec, vec, vec],
        out_specs=pl.BlockSpec((blk, dp), lambda i: (i, 0)),
        out_shape=jax.ShapeDtypeStruct((n, dp), jnp.float32),
    )(agg, nd, b.reshape(1, dp), g.reshape(1, dp), be.reshape(1, dp))


def kernel(features, W1, b1, g1, be1, W2, b2, g2, be2, W3, b3, g3, be3,
           edge_index):
    n = features.shape[0]
    src = edge_index[0]
    dst = edge_index[1]
    n_cls = W3.shape[1]
    dp3 = 128
    w3p = jnp.pad(W3, ((0, 0), (0, dp3 - n_cls)))
    b3p = jnp.pad(b3, (0, dp3 - n_cls))
    g3p = jnp.pad(g3, (0, dp3 - n_cls))
    be3p = jnp.pad(be3, (0, dp3 - n_cls))

    hs, hd = _sc_hist(src, dst, n)
    ns, nd = _tc_norms(hs, hd)

    def agg_of(xw):
        return _sc_aggregate(xw, src, dst)

    xw1 = _tc_matmul(features, ns, W1)
    agg1 = agg_of(xw1)
    xw2 = _tc_postmm(agg1, nd, b1, g1, be1, ns, W2, 128)
    agg2 = agg_of(xw2)
    xw3 = _tc_postmm(agg2, nd, b2, g2, be2, ns, w3p, 128)
    agg3 = agg_of(xw3)
    h = _tc_post(agg3, nd, b3p, g3p, be3p, n_cls)
    return h[:, :n_cls]
